# baseline (speedup 1.0000x reference)
import functools

import jax
import jax.numpy as jnp
from jax import lax
from jax.experimental import pallas as pl
from jax.experimental.pallas import tpu as pltpu
from jax.experimental.pallas import tpu_sc as plsc

EPS = 1e-12

M_BLK = 512
N_BLK = 1024


def _argmin_body(hn_ref, ent_ref, h2_ref, e2_ref, idx_ref, best_d, best_i):
    j = pl.program_id(1)
    s = lax.dot_general(hn_ref[...], ent_ref[...], (((1,), (0,)), ((), ())),
                        preferred_element_type=jnp.float32)
    d = h2_ref[...] + e2_ref[...] - 2.0 * s
    bmin = jnp.min(d, axis=1, keepdims=True)
    barg = jnp.argmin(d, axis=1).astype(jnp.int32)[:, None] + j * N_BLK

    @pl.when(j == 0)
    def _init():
        best_d[...] = bmin
        best_i[...] = barg

    @pl.when(j > 0)
    def _update():
        upd = bmin < best_d[...]
        best_d[...] = jnp.where(upd, bmin, best_d[...])
        best_i[...] = jnp.where(upd, barg, best_i[...])

    @pl.when(j == pl.num_programs(1) - 1)
    def _emit():
        idx_ref[...] = best_i[...]


def _loss_body(zq_ref, hn_ref, zqn_ref, loss_ref):
    zq = zq_ref[...]
    zqn = zq * lax.rsqrt(jnp.sum(zq * zq, axis=1, keepdims=True) + EPS)
    zqn_ref[...] = zqn
    diff = zqn - hn_ref[...]
    loss_ref[...] = (jnp.sum(diff * diff, keepdims=True)
                     / (zq.shape[0] * zq.shape[1]))


def _sc_gather(table, idx):
    n_tok = idx.shape[0]
    dim = table.shape[1]
    info = plsc.get_sparse_core_info()
    nw = info.num_cores * info.num_subcores
    b_per_w = n_tok // nw
    chunk = 128
    n_chunks = b_per_w // chunk
    idx_2d = idx.reshape(n_tok // chunk, chunk)
    mesh = plsc.VectorSubcoreMesh(core_axis_name="c", subcore_axis_name="s")

    @functools.partial(
        pl.kernel,
        mesh=mesh,
        compiler_params=pltpu.CompilerParams(use_tc_tiling_on_sc=False),
        out_type=jax.ShapeDtypeStruct((n_tok, dim), jnp.float32),
        scratch_types=[
            pltpu.VMEM((n_chunks, chunk), jnp.int32),
            pltpu.VMEM((b_per_w, dim), jnp.float32),
            pltpu.SemaphoreType.DMA,
        ],
    )
    def gather_k(table_hbm, idx_hbm, out_hbm, idx_v, rows_v, sem):
        wid = lax.axis_index("s") * info.num_cores + lax.axis_index("c")
        base = wid * b_per_w
        pltpu.sync_copy(idx_hbm.at[pl.ds(wid * n_chunks, n_chunks)], idx_v)
        copies = []
        for c in range(n_chunks):
            copies.append(pltpu.async_copy(
                table_hbm.at[idx_v.at[c]],
                rows_v.at[pl.ds(c * chunk, chunk)],
                sem,
            ))
        for cp in copies:
            cp.wait()
        pltpu.sync_copy(rows_v, out_hbm.at[pl.ds(base, b_per_w)])

    return gather_k(table, idx_2d)


def kernel(hidden_states, embedding):
    b, t, dim = hidden_states.shape
    n_tok = b * t
    n_codes = embedding.shape[0]
    h = hidden_states.reshape(n_tok, dim)
    hn = h * lax.rsqrt(jnp.sum(h * h, axis=1, keepdims=True) + EPS)
    en = embedding * lax.rsqrt(
        jnp.sum(embedding * embedding, axis=1, keepdims=True) + EPS)
    h2 = jnp.sum(hn * hn, axis=1, keepdims=True)
    e2 = jnp.sum(en * en, axis=1)[None, :]
    ent = en.T

    grid = (n_tok // M_BLK, n_codes // N_BLK)
    idx2 = pl.pallas_call(
        _argmin_body,
        grid=grid,
        in_specs=[
            pl.BlockSpec((M_BLK, dim), lambda i, j: (i, 0)),
            pl.BlockSpec((dim, N_BLK), lambda i, j: (0, j)),
            pl.BlockSpec((M_BLK, 1), lambda i, j: (i, 0)),
            pl.BlockSpec((1, N_BLK), lambda i, j: (0, j)),
        ],
        out_specs=pl.BlockSpec((M_BLK, 1), lambda i, j: (i, 0)),
        out_shape=jax.ShapeDtypeStruct((n_tok, 1), jnp.int32),
        scratch_shapes=[
            pltpu.VMEM((M_BLK, 1), jnp.float32),
            pltpu.VMEM((M_BLK, 1), jnp.int32),
        ],
    )(hn, ent, h2, e2)
    idx = idx2.reshape(n_tok)

    zq_raw = _sc_gather(embedding, idx)

    zqn, loss = pl.pallas_call(
        _loss_body,
        out_shape=[
            jax.ShapeDtypeStruct((n_tok, dim), jnp.float32),
            jax.ShapeDtypeStruct((1, 1), jnp.float32),
        ],
    )(zq_raw, hn)

    loss_s = loss[0, 0]
    return (zqn.reshape(b, t, dim), idx2.reshape(b, t),
            (loss_s, loss_s))
